# Initial kernel scaffold; baseline (speedup 1.0000x reference)
import jax
import jax.numpy as jnp
from jax.experimental import pallas as pl


def kernel(virtual_node, embeddings, batch_vector, W1, b1, g1, be1, W2, b2, g2, be2):
    raise NotImplementedError("write your pallas kernel here")


# speedup vs baseline: 6.5016x; 6.5016x over previous
import functools

import jax
import jax.numpy as jnp
from jax.experimental import pallas as pl

NUM_GRAPHS = 512
N_NODES = 100000
DIM = 128

SEG_BLOCK = 2000
NUM_BLOCKS = N_NODES // SEG_BLOCK


def _segsum_body(bv_ref, emb_ref, out_ref):
    i = pl.program_id(0)

    @pl.when(i == 0)
    def _():
        out_ref[...] = jnp.zeros_like(out_ref)

    seg = bv_ref[0, 0, :]
    iota = jax.lax.broadcasted_iota(jnp.int32, (NUM_GRAPHS, SEG_BLOCK), 0)
    onehot = (seg[None, :] == iota).astype(jnp.float32)
    out_ref[...] += jax.lax.dot_general(
        onehot, emb_ref[...],
        dimension_numbers=(((1,), (0,)), ((), ())),
        preferred_element_type=jnp.float32,
    )


def _mlp_body(g_ref, vn_ref, w1_ref, b1_ref, g1_ref, be1_ref,
              w2_ref, b2_ref, g2_ref, be2_ref, out_ref):
    vn = vn_ref[...] + g_ref[...]
    h = jax.lax.dot_general(
        vn, w1_ref[...], dimension_numbers=(((1,), (1,)), ((), ())),
        preferred_element_type=jnp.float32,
    ) + b1_ref[...]
    mu = jnp.mean(h, axis=0, keepdims=True)
    var = jnp.mean((h - mu) ** 2, axis=0, keepdims=True)
    h = g1_ref[...] * (h - mu) * jax.lax.rsqrt(var + 1e-5) + be1_ref[...]
    h = jnp.maximum(h, 0.0)
    h = jax.lax.dot_general(
        h, w2_ref[...], dimension_numbers=(((1,), (1,)), ((), ())),
        preferred_element_type=jnp.float32,
    ) + b2_ref[...]
    mu2 = jnp.mean(h, axis=0, keepdims=True)
    var2 = jnp.mean((h - mu2) ** 2, axis=0, keepdims=True)
    h = g2_ref[...] * (h - mu2) * jax.lax.rsqrt(var2 + 1e-5) + be2_ref[...]
    out_ref[...] = jnp.maximum(h, 0.0)


@functools.partial(jax.jit, static_argnames=("interpret",))
def kernel(virtual_node, embeddings, batch_vector, W1, b1, g1, be1,
           W2, b2, g2, be2, interpret=False):
    bv = batch_vector.astype(jnp.int32).reshape(NUM_BLOCKS, 1, SEG_BLOCK)

    seg_sums = pl.pallas_call(
        _segsum_body,
        grid=(NUM_BLOCKS,),
        in_specs=[
            pl.BlockSpec((1, 1, SEG_BLOCK), lambda i: (i, 0, 0)),
            pl.BlockSpec((SEG_BLOCK, DIM), lambda i: (i, 0)),
        ],
        out_specs=pl.BlockSpec((NUM_GRAPHS, DIM), lambda i: (0, 0)),
        out_shape=jax.ShapeDtypeStruct((NUM_GRAPHS, DIM), jnp.float32),
        interpret=interpret,
    )(bv, embeddings)

    full = lambda s: pl.BlockSpec(s, lambda: (0,) * len(s))
    out = pl.pallas_call(
        _mlp_body,
        in_specs=[
            full((NUM_GRAPHS, DIM)), full((NUM_GRAPHS, DIM)),
            full((2 * DIM, DIM)), full((1, 2 * DIM)), full((1, 2 * DIM)),
            full((1, 2 * DIM)),
            full((DIM, 2 * DIM)), full((1, DIM)), full((1, DIM)),
            full((1, DIM)),
        ],
        out_specs=full((NUM_GRAPHS, DIM)),
        out_shape=jax.ShapeDtypeStruct((NUM_GRAPHS, DIM), jnp.float32),
        interpret=interpret,
    )(seg_sums, virtual_node, W1, b1.reshape(1, -1), g1.reshape(1, -1),
      be1.reshape(1, -1), W2, b2.reshape(1, -1), g2.reshape(1, -1),
      be2.reshape(1, -1))
    return out
